# baseline (speedup 1.0000x reference)
import functools

import jax
import jax.numpy as jnp
from jax import lax
from jax.experimental import pallas as pl
from jax.experimental.pallas import tpu as pltpu
from jax.experimental.pallas import tpu_sc as plsc

_B, _T = 4096, 200
_V, _D = 1000000, 32
_NW = 32
_BB = 128
_NB = 4


def _build():
  mesh = plsc.VectorSubcoreMesh(core_axis_name="c", subcore_axis_name="s")
  per_w = _BB * _T

  @functools.partial(
      pl.kernel,
      mesh=mesh,
      out_type=jax.ShapeDtypeStruct((_T, 4, _NW, 8, 128), jnp.float32),
      scratch_types=[
          pltpu.VMEM((per_w,), jnp.int32),
          pltpu.VMEM((_NB, _BB, _D), jnp.float32),
          pltpu.VMEM((_NB, 4, 8, 128), jnp.float32),
          pltpu.VMEM((_NB, _BB), jnp.int32),
          pltpu.SemaphoreType.DMA((_NB,)),
          pltpu.SemaphoreType.DMA((_NB,)),
      ],
      compiler_params=pltpu.CompilerParams(
          use_tc_tiling_on_sc=False, needs_layout_passes=False),
  )
  def emb(idx_hbm, w_hbm, out_hbm, idx_v, g_v, tr_v, il_v, gsem, ssem):
    wid = lax.axis_index("s") * 2 + lax.axis_index("c")
    pltpu.sync_copy(idx_hbm.at[pl.ds(wid * per_w, per_w)], idx_v)
    lanes = lax.iota(jnp.int32, 16)

    def gcp(b):
      return pltpu.make_async_copy(
          w_hbm.at[il_v.at[b]], g_v.at[b], gsem.at[b])

    def scp(t, b):
      return pltpu.make_async_copy(
          tr_v.at[b], out_hbm.at[t, :, wid], ssem.at[b])

    def extract(t, b):
      for k in range(8):
        pos = (k * 16 + lanes) * _T + t
        il_v[b, pl.ds(k * 16, 16)] = plsc.load_gather(idx_v, [pos])

    def transpose(b):
      def body(d0, carry):
        dvec = jnp.bitwise_and(d0 + lanes, _D - 1)
        i0 = lax.shift_right_logical(dvec, 3)
        i1 = jnp.bitwise_and(dvec, 7)
        for k in range(8):
          jvec = k * 16 + lanes
          vals = plsc.load_gather(g_v.at[b], [jvec, dvec])
          plsc.store_scatter(tr_v.at[b], [i0, i1, jvec], vals)
        return carry
      lax.fori_loop(0, _D, body, 0)

    for b in range(_NB):
      extract(b, b)
      gcp(b).start()

    def main_body(r, carry):
      for b in range(_NB):
        t_new = r * _NB + b
        t_old = t_new - _NB
        gcp(b).wait()

        @pl.when(r >= 2)
        def _():
          scp(t_old, b).wait()

        transpose(b)
        scp(t_old, b).start()
        extract(t_new, b)
        gcp(b).start()
      return carry

    lax.fori_loop(1, _T // _NB, main_body, 0)

    for b in range(_NB):
      t_old = _T - _NB + b
      gcp(b).wait()
      scp(t_old, b).wait()
      transpose(b)
      scp(t_old, b).start()
    for b in range(_NB):
      scp(_T - _NB + b, b).wait()

  return emb


@jax.jit
def kernel(x, weight):
  flat = x.reshape(-1)
  out5 = _build()(flat, weight)
  return out5.transpose(2, 4, 0, 1, 3).reshape(_B, _T, _D)
